# baseline (speedup 1.0000x reference)
import functools

import jax
import jax.numpy as jnp
from jax import lax
from jax.experimental import pallas as pl
from jax.experimental.pallas import tpu as pltpu
from jax.experimental.pallas import tpu_sc as plsc

_N = 10000
_NP = 10240
_E = 320000
_D = 128
_NC = 2
_NS = 16
_NW = _NC * _NS
_K = 40
_CHUNKS = 256
_EPW = _K * _CHUNKS
_EPAD = _EPW * _NW
_RPS = _NP // _NS

_mesh = plsc.VectorSubcoreMesh(core_axis_name="c", subcore_axis_name="s")


def _splat(v, l):
    idx = jnp.full((16, 1), l, jnp.int32)
    dn = lax.GatherDimensionNumbers(
        offset_dims=(), collapsed_slice_dims=(0,), start_index_map=(0,))
    return lax.gather(v, idx, dn, (1,),
                      mode=lax.GatherScatterMode.PROMISE_IN_BOUNDS)


@functools.partial(
    pl.kernel,
    out_type=jax.ShapeDtypeStruct((_NC, _NP), jnp.float32),
    mesh=_mesh,
    scratch_types=(
        [pltpu.VMEM((_K,), jnp.int32) for _ in range(8)]
        + [pltpu.VMEM((_K,), jnp.float32) for _ in range(8)]
        + [pltpu.VMEM((_RPS,), jnp.float32)]
        + [pltpu.VMEM_SHARED((_NP,), jnp.float32)]
        + [pltpu.SemaphoreType.DMA for _ in range(12)]
    ),
)
def _deg_kernel(col_hbm, ew_hbm, out_hbm, *refs):
    colq = refs[0:8]
    ewq = refs[8:16]
    zb = refs[16]
    dacc = refs[17]
    seme = refs[18:26]
    semsc = refs[26:30]
    c = lax.axis_index("c")
    s = lax.axis_index("s")
    wid = c * _NS + s

    def zrow(i, carry):
        zb[pl.ds(i * 16, 16)] = jnp.zeros((16,), jnp.float32)
        return carry
    lax.fori_loop(0, _RPS // 16, zrow, 0)

    rbase = s * _RPS
    pltpu.sync_copy(zb, dacc.at[pl.ds(rbase, _RPS)])
    plsc.subcore_barrier()

    def e_desc(ph, i):
        eb = wid * _EPW + i * _K
        return (pltpu.make_async_copy(col_hbm.at[pl.ds(eb, _K)], colq[ph],
                                      seme[ph]),
                pltpu.make_async_copy(ew_hbm.at[pl.ds(eb, _K)], ewq[ph],
                                      seme[ph]))

    def e_start(ph, i):
        for d in e_desc(ph, i):
            d.start()

    def e_wait(ph, i):
        for d in e_desc(ph, i):
            d.wait()

    def s_desc(ph, k):
        return pltpu.make_async_copy(ewq[ph], dacc.at[colq[ph]], semsc[k])

    def phase(i8, ph, head=False, tail=False):
        i = i8 * 8 + ph
        k = ph % 4
        if not (head and ph < 4):
            s_desc((ph + 4) % 8, k).wait()
        e_wait(ph, i)
        pltpu.async_copy(ewq[ph], dacc.at[colq[ph]], semsc[k], add=True)
        if not (tail and ph > 4):
            e_start((ph + 3) % 8, i + 3)

    for ph in range(3):
        e_start(ph, ph)
    for ph in range(8):
        phase(0, ph, head=True)

    def body(i8, carry):
        for ph in range(8):
            phase(i8, ph)
        return carry
    lax.fori_loop(1, _CHUNKS // 8 - 1, body, 0)
    for ph in range(8):
        phase(_CHUNKS // 8 - 1, ph, tail=True)
    for k in range(4):
        s_desc(k + 4, k).wait()
    plsc.subcore_barrier()
    pltpu.sync_copy(dacc.at[pl.ds(rbase, _RPS)],
                    out_hbm.at[c, pl.ds(rbase, _RPS)])


_NPAIR = _NP // 2
_PPS = _NPAIR // _NS
_EPT = _EPAD // _NS
_CHP = _EPT // _K


_DH = _D // 2


@functools.partial(
    pl.kernel,
    out_type=jax.ShapeDtypeStruct((_NC, _NPAIR, _D), jnp.float32),
    mesh=_mesh,
    scratch_types=(
        [pltpu.VMEM((_K,), jnp.int32) for _ in range(8)]
        + [pltpu.VMEM((_K,), jnp.int32) for _ in range(8)]
        + [pltpu.VMEM((_K,), jnp.float32) for _ in range(8)]
        + [pltpu.VMEM((_K,), jnp.float32) for _ in range(8)]
        + [pltpu.VMEM((_K,), jnp.float32) for _ in range(8)]
        + [pltpu.VMEM((_K, _D), jnp.float32) for _ in range(4)]
        + [pltpu.VMEM_SHARED((_NPAIR, _D), jnp.float32)]
        + [pltpu.VMEM_SHARED((_NPAIR, _D), jnp.float32)]
        + [pltpu.SemaphoreType.DMA for _ in range(16)]
    ),
)
def _prop_kernel(y_hbm, row_hbm, col_hbm, ew_hbm, rp_hbm, cp_hbm,
                 out_hbm, *refs):
    rowq = refs[0:8]
    colq = refs[8:16]
    ewq = refs[16:24]
    rpq = refs[24:32]
    cpq = refs[32:40]
    rows = refs[40:44]
    ys = refs[44]
    acc = refs[45]
    seme = refs[46:54]
    semg = refs[54:58]
    semsc = refs[58:62]
    c = lax.axis_index("c")
    s = lax.axis_index("s")

    def zrow(i, carry):
        for j in range(_D // 16):
            rows[0][i, pl.ds(j * 16, 16)] = jnp.zeros((16,), jnp.float32)
        return carry
    lax.fori_loop(0, _K, zrow, 0)

    rbase = s * _PPS
    for k in range(_PPS // _K):
        pltpu.sync_copy(rows[0], acc.at[pl.ds(rbase + k * _K, _K)])
    pltpu.sync_copy(y_hbm.at[c, pl.ds(rbase, _PPS)],
                    ys.at[pl.ds(rbase, _PPS)])
    plsc.subcore_barrier()

    def e_desc(ph, i):
        eb = s * _EPT + i * _K
        return (pltpu.make_async_copy(row_hbm.at[pl.ds(eb, _K)], rowq[ph],
                                      seme[ph]),
                pltpu.make_async_copy(col_hbm.at[pl.ds(eb, _K)], colq[ph],
                                      seme[ph]),
                pltpu.make_async_copy(ew_hbm.at[pl.ds(eb, _K)], ewq[ph],
                                      seme[ph]),
                pltpu.make_async_copy(rp_hbm.at[pl.ds(eb, _K)], rpq[ph],
                                      seme[ph]),
                pltpu.make_async_copy(cp_hbm.at[pl.ds(eb, _K)], cpq[ph],
                                      seme[ph]))

    def e_start(ph, i):
        for d in e_desc(ph, i):
            d.start()

    def e_wait(ph, i):
        for d in e_desc(ph, i):
            d.wait()

    def g_desc(ph, k):
        return pltpu.make_async_copy(ys.at[rowq[ph]], rows[k], semg[k])

    def s_desc(ph, k):
        return pltpu.make_async_copy(rows[k], acc.at[colq[ph]], semsc[k])

    def scale(k, ph):
        one = jnp.ones((16,), jnp.float32)

        def grp(g, cc):
            ew_v = ewq[ph][pl.ds(g * 16, 16)]
            rp_v = rpq[ph][pl.ds(g * 16, 16)]
            cp_v = cpq[ph][pl.ds(g * 16, 16)]
            elo = ew_v * (one - cp_v)
            ehi = ew_v * cp_v
            a_v = elo * (one - rp_v)
            b_v = elo * rp_v
            c_v = ehi * (one - rp_v)
            d_v = ehi * rp_v
            for l in range(16):
                sa = _splat(a_v, l)
                sb = _splat(b_v, l)
                sc = _splat(c_v, l)
                sd = _splat(d_v, l)
                r = g * 16 + l
                for j in range(_D // 32):
                    vlo = rows[k][r, pl.ds(j * 16, 16)]
                    vhi = rows[k][r, pl.ds(64 + j * 16, 16)]
                    rows[k][r, pl.ds(j * 16, 16)] = vlo * sa + vhi * sb
                    rows[k][r, pl.ds(64 + j * 16, 16)] = vlo * sc + vhi * sd
            return cc
        lax.fori_loop(0, _K // 16, grp, 0)

    def phase(i8, ph, head=False, tail=False):
        i = i8 * 8 + ph
        k = ph % 4
        kp = (ph - 1) % 4
        pp = (ph - 1) % 8
        if not (head and ph < 4):
            s_desc((ph + 4) % 8, k).wait()
        e_wait(ph, i)
        g_desc(ph, k).start()
        if not (tail and ph > 4):
            e_start((ph + 3) % 8, i + 3)
        if not (head and ph == 0):
            g_desc(pp, kp).wait()
            scale(kp, pp)
            pltpu.async_copy(rows[kp], acc.at[colq[pp]], semsc[kp],
                             add=True)

    for ph in range(3):
        e_start(ph, ph)
    for ph in range(8):
        phase(0, ph, head=True)

    def body(i8, carry):
        for ph in range(8):
            phase(i8, ph)
        return carry
    lax.fori_loop(1, _CHP // 8 - 1, body, 0)
    for ph in range(8):
        phase(_CHP // 8 - 1, ph, tail=True)
    g_desc(7, 3).wait()
    scale(3, 7)
    pltpu.async_copy(rows[3], acc.at[colq[7]], semsc[3], add=True)
    for k in range(4):
        s_desc(k + 4, k).wait()
    plsc.subcore_barrier()
    pltpu.sync_copy(acc.at[pl.ds(rbase, _PPS)],
                    out_hbm.at[c, pl.ds(rbase, _PPS)])


_R = 1024


def _mm_body(x_ref, w_ref, d_ref, y_ref, xw_ref):
    xw = jnp.dot(x_ref[...], w_ref[...], preferred_element_type=jnp.float32)
    deg = d_ref[0, :] + d_ref[1, :] + 1.0
    dis = jnp.where(deg > 0, lax.rsqrt(deg), 0.0)
    xw_ref[...] = xw
    y_ref[...] = xw * dis[:, None]


def _mm_call(x, W, dpart):
    return pl.pallas_call(
        _mm_body,
        grid=(_NP // _R,),
        in_specs=[
            pl.BlockSpec((_R, _D), lambda i: (i, 0)),
            pl.BlockSpec((_D, _D), lambda i: (0, 0)),
            pl.BlockSpec((_NC, _R), lambda i: (0, i)),
        ],
        out_specs=[
            pl.BlockSpec((_R, _D), lambda i: (i, 0)),
            pl.BlockSpec((_R, _D), lambda i: (i, 0)),
        ],
        out_shape=[
            jax.ShapeDtypeStruct((_NP, _D), jnp.float32),
            jax.ShapeDtypeStruct((_N, _D), jnp.float32),
        ],
    )(x, W, dpart)


def _fin_body(a_ref, d_ref, xw_ref, b_ref, o_ref):
    deg = d_ref[0, :] + d_ref[1, :] + 1.0
    dis = jnp.where(deg > 0, lax.rsqrt(deg), 0.0)
    inv = jnp.where(deg > 0, 1.0 / deg, 0.0)
    o_ref[...] = (a_ref[...] * dis[:, None]
                  + xw_ref[...] * inv[:, None] + b_ref[...])


def _fin_call(accf, dpart, xw, b2):
    return pl.pallas_call(
        _fin_body,
        grid=(_NP // _R,),
        in_specs=[
            pl.BlockSpec((_R, _D), lambda i: (i, 0)),
            pl.BlockSpec((_NC, _R), lambda i: (0, i)),
            pl.BlockSpec((_R, _D), lambda i: (i, 0)),
            pl.BlockSpec((1, _D), lambda i: (0, 0)),
        ],
        out_specs=pl.BlockSpec((_R, _D), lambda i: (i, 0)),
        out_shape=jax.ShapeDtypeStruct((_N, _D), jnp.float32),
    )(accf, dpart, xw, b2)


def kernel(x, edge_index, edge_weight, W, b):
    row = edge_index[0].astype(jnp.int32)
    col = edge_index[1].astype(jnp.int32)
    ew = edge_weight.astype(jnp.float32)
    pad = _EPAD - _E
    row_p = jnp.pad(row, (0, pad))
    col_p = jnp.pad(col, (0, pad))
    ew_p = jnp.pad(ew, (0, pad))

    dpart = _deg_kernel(col_p, ew_p)
    y, xw = _mm_call(x, W, dpart)
    y2 = jnp.stack([y[:, :_DH], y[:, _DH:]],
                   axis=0).reshape(_NC, _NPAIR, _D)
    rp_p = (row_p & 1).astype(jnp.float32)
    cp_p = (col_p & 1).astype(jnp.float32)
    acc2 = _prop_kernel(y2, row_p >> 1, col_p >> 1, ew_p, rp_p, cp_p)
    accf = acc2.reshape(_NC, _NPAIR, 2, _DH).transpose(
        (1, 2, 0, 3)).reshape(_NP, _D)
    return _fin_call(accf, dpart, xw, b.reshape(1, _D))


# speedup vs baseline: 1.0760x; 1.0760x over previous
import functools

import jax
import jax.numpy as jnp
from jax import lax
from jax.experimental import pallas as pl
from jax.experimental.pallas import tpu as pltpu
from jax.experimental.pallas import tpu_sc as plsc

_N = 10000
_NP = 10240
_E = 320000
_D = 128
_NC = 2
_NS = 16
_NW = _NC * _NS
_K = 80
_CHUNKS = 128
_EPW = _K * _CHUNKS
_EPAD = _EPW * _NW
_RPS = _NP // _NS

_mesh = plsc.VectorSubcoreMesh(core_axis_name="c", subcore_axis_name="s")


def _splat(v, l):
    idx = jnp.full((16, 1), l, jnp.int32)
    dn = lax.GatherDimensionNumbers(
        offset_dims=(), collapsed_slice_dims=(0,), start_index_map=(0,))
    return lax.gather(v, idx, dn, (1,),
                      mode=lax.GatherScatterMode.PROMISE_IN_BOUNDS)


@functools.partial(
    pl.kernel,
    out_type=jax.ShapeDtypeStruct((_NC, _NP), jnp.float32),
    mesh=_mesh,
    scratch_types=(
        [pltpu.VMEM((_K,), jnp.int32) for _ in range(8)]
        + [pltpu.VMEM((_K,), jnp.float32) for _ in range(8)]
        + [pltpu.VMEM((_RPS,), jnp.float32)]
        + [pltpu.VMEM_SHARED((_NP,), jnp.float32)]
        + [pltpu.SemaphoreType.DMA for _ in range(12)]
    ),
)
def _deg_kernel(col_hbm, ew_hbm, out_hbm, *refs):
    colq = refs[0:8]
    ewq = refs[8:16]
    zb = refs[16]
    dacc = refs[17]
    seme = refs[18:26]
    semsc = refs[26:30]
    c = lax.axis_index("c")
    s = lax.axis_index("s")
    wid = c * _NS + s

    def zrow(i, carry):
        zb[pl.ds(i * 16, 16)] = jnp.zeros((16,), jnp.float32)
        return carry
    lax.fori_loop(0, _RPS // 16, zrow, 0)

    rbase = s * _RPS
    pltpu.sync_copy(zb, dacc.at[pl.ds(rbase, _RPS)])
    plsc.subcore_barrier()

    def e_desc(ph, i):
        eb = wid * _EPW + i * _K
        return (pltpu.make_async_copy(col_hbm.at[pl.ds(eb, _K)], colq[ph],
                                      seme[ph]),
                pltpu.make_async_copy(ew_hbm.at[pl.ds(eb, _K)], ewq[ph],
                                      seme[ph]))

    def e_start(ph, i):
        for d in e_desc(ph, i):
            d.start()

    def e_wait(ph, i):
        for d in e_desc(ph, i):
            d.wait()

    def s_desc(ph, k):
        return pltpu.make_async_copy(ewq[ph], dacc.at[colq[ph]], semsc[k])

    def phase(i8, ph, head=False, tail=False):
        i = i8 * 8 + ph
        k = ph % 4
        if not (head and ph < 4):
            s_desc((ph + 4) % 8, k).wait()
        e_wait(ph, i)
        pltpu.async_copy(ewq[ph], dacc.at[colq[ph]], semsc[k], add=True)
        if not (tail and ph > 4):
            e_start((ph + 3) % 8, i + 3)

    for ph in range(3):
        e_start(ph, ph)
    for ph in range(8):
        phase(0, ph, head=True)

    def body(i8, carry):
        for ph in range(8):
            phase(i8, ph)
        return carry
    lax.fori_loop(1, _CHUNKS // 8 - 1, body, 0)
    for ph in range(8):
        phase(_CHUNKS // 8 - 1, ph, tail=True)
    for k in range(4):
        s_desc(k + 4, k).wait()
    plsc.subcore_barrier()
    pltpu.sync_copy(dacc.at[pl.ds(rbase, _RPS)],
                    out_hbm.at[c, pl.ds(rbase, _RPS)])


_NPAIR = _NP // 2
_PPS = _NPAIR // _NS
_EPT = _EPAD // _NS
_CHP = _EPT // _K


_DH = _D // 2


@functools.partial(
    pl.kernel,
    out_type=jax.ShapeDtypeStruct((_NC, _NPAIR, _D), jnp.float32),
    mesh=_mesh,
    scratch_types=(
        [pltpu.VMEM((_K,), jnp.int32) for _ in range(8)]
        + [pltpu.VMEM((_K,), jnp.int32) for _ in range(8)]
        + [pltpu.VMEM((_K,), jnp.float32) for _ in range(8)]
        + [pltpu.VMEM((_K,), jnp.float32) for _ in range(8)]
        + [pltpu.VMEM((_K,), jnp.float32) for _ in range(8)]
        + [pltpu.VMEM((_K, _D), jnp.float32) for _ in range(4)]
        + [pltpu.VMEM_SHARED((_NPAIR, _D), jnp.float32)]
        + [pltpu.VMEM_SHARED((_NPAIR, _D), jnp.float32)]
        + [pltpu.SemaphoreType.DMA for _ in range(16)]
    ),
)
def _prop_kernel(y_hbm, row_hbm, col_hbm, ew_hbm, rp_hbm, cp_hbm,
                 out_hbm, *refs):
    rowq = refs[0:8]
    colq = refs[8:16]
    ewq = refs[16:24]
    rpq = refs[24:32]
    cpq = refs[32:40]
    rows = refs[40:44]
    ys = refs[44]
    acc = refs[45]
    seme = refs[46:54]
    semg = refs[54:58]
    semsc = refs[58:62]
    c = lax.axis_index("c")
    s = lax.axis_index("s")

    def zrow(i, carry):
        for j in range(_D // 16):
            rows[0][i, pl.ds(j * 16, 16)] = jnp.zeros((16,), jnp.float32)
        return carry
    lax.fori_loop(0, _K, zrow, 0)

    rbase = s * _PPS
    for k in range(_PPS // _K):
        pltpu.sync_copy(rows[0], acc.at[pl.ds(rbase + k * _K, _K)])
    pltpu.sync_copy(y_hbm.at[c, pl.ds(rbase, _PPS)],
                    ys.at[pl.ds(rbase, _PPS)])
    plsc.subcore_barrier()

    def e_desc(ph, i):
        eb = s * _EPT + i * _K
        return (pltpu.make_async_copy(row_hbm.at[pl.ds(eb, _K)], rowq[ph],
                                      seme[ph]),
                pltpu.make_async_copy(col_hbm.at[pl.ds(eb, _K)], colq[ph],
                                      seme[ph]),
                pltpu.make_async_copy(ew_hbm.at[pl.ds(eb, _K)], ewq[ph],
                                      seme[ph]),
                pltpu.make_async_copy(rp_hbm.at[pl.ds(eb, _K)], rpq[ph],
                                      seme[ph]),
                pltpu.make_async_copy(cp_hbm.at[pl.ds(eb, _K)], cpq[ph],
                                      seme[ph]))

    def e_start(ph, i):
        for d in e_desc(ph, i):
            d.start()

    def e_wait(ph, i):
        for d in e_desc(ph, i):
            d.wait()

    def g_desc(ph, k):
        return pltpu.make_async_copy(ys.at[rowq[ph]], rows[k], semg[k])

    def s_desc(ph, k):
        return pltpu.make_async_copy(rows[k], acc.at[colq[ph]], semsc[k])

    def scale(k, ph):
        one = jnp.ones((16,), jnp.float32)

        def grp(g, cc):
            ew_v = ewq[ph][pl.ds(g * 16, 16)]
            rp_v = rpq[ph][pl.ds(g * 16, 16)]
            cp_v = cpq[ph][pl.ds(g * 16, 16)]
            elo = ew_v * (one - cp_v)
            ehi = ew_v * cp_v
            a_v = elo * (one - rp_v)
            b_v = elo * rp_v
            c_v = ehi * (one - rp_v)
            d_v = ehi * rp_v
            for l in range(16):
                sa = _splat(a_v, l)
                sb = _splat(b_v, l)
                sc = _splat(c_v, l)
                sd = _splat(d_v, l)
                r = g * 16 + l
                for j in range(_D // 32):
                    vlo = rows[k][r, pl.ds(j * 16, 16)]
                    vhi = rows[k][r, pl.ds(64 + j * 16, 16)]
                    rows[k][r, pl.ds(j * 16, 16)] = vlo * sa + vhi * sb
                    rows[k][r, pl.ds(64 + j * 16, 16)] = vlo * sc + vhi * sd
            return cc
        lax.fori_loop(0, _K // 16, grp, 0)

    def phase(i8, ph, head=False, tail=False):
        i = i8 * 8 + ph
        k = ph % 4
        kp = (ph - 1) % 4
        pp = (ph - 1) % 8
        if not (head and ph < 4):
            s_desc((ph + 4) % 8, k).wait()
        e_wait(ph, i)
        g_desc(ph, k).start()
        if not (tail and ph > 4):
            e_start((ph + 3) % 8, i + 3)
        if not (head and ph == 0):
            g_desc(pp, kp).wait()
            scale(kp, pp)
            pltpu.async_copy(rows[kp], acc.at[colq[pp]], semsc[kp],
                             add=True)

    for ph in range(3):
        e_start(ph, ph)
    for ph in range(8):
        phase(0, ph, head=True)

    def body(i8, carry):
        for ph in range(8):
            phase(i8, ph)
        return carry
    lax.fori_loop(1, _CHP // 8 - 1, body, 0)
    for ph in range(8):
        phase(_CHP // 8 - 1, ph, tail=True)
    g_desc(7, 3).wait()
    scale(3, 7)
    pltpu.async_copy(rows[3], acc.at[colq[7]], semsc[3], add=True)
    for k in range(4):
        s_desc(k + 4, k).wait()
    plsc.subcore_barrier()
    pltpu.sync_copy(acc.at[pl.ds(rbase, _PPS)],
                    out_hbm.at[c, pl.ds(rbase, _PPS)])


_R = 1024


def _mm_body(x_ref, w_ref, d_ref, y_ref, xw_ref):
    xw = jnp.dot(x_ref[...], w_ref[...], preferred_element_type=jnp.float32)
    deg = d_ref[0, :] + d_ref[1, :] + 1.0
    dis = jnp.where(deg > 0, lax.rsqrt(deg), 0.0)
    xw_ref[...] = xw
    y_ref[...] = xw * dis[:, None]


def _mm_call(x, W, dpart):
    return pl.pallas_call(
        _mm_body,
        grid=(_NP // _R,),
        in_specs=[
            pl.BlockSpec((_R, _D), lambda i: (i, 0)),
            pl.BlockSpec((_D, _D), lambda i: (0, 0)),
            pl.BlockSpec((_NC, _R), lambda i: (0, i)),
        ],
        out_specs=[
            pl.BlockSpec((_R, _D), lambda i: (i, 0)),
            pl.BlockSpec((_R, _D), lambda i: (i, 0)),
        ],
        out_shape=[
            jax.ShapeDtypeStruct((_NP, _D), jnp.float32),
            jax.ShapeDtypeStruct((_N, _D), jnp.float32),
        ],
    )(x, W, dpart)


def _fin_body(a_ref, d_ref, xw_ref, b_ref, o_ref):
    deg = d_ref[0, :] + d_ref[1, :] + 1.0
    dis = jnp.where(deg > 0, lax.rsqrt(deg), 0.0)
    inv = jnp.where(deg > 0, 1.0 / deg, 0.0)
    o_ref[...] = (a_ref[...] * dis[:, None]
                  + xw_ref[...] * inv[:, None] + b_ref[...])


def _fin_call(accf, dpart, xw, b2):
    return pl.pallas_call(
        _fin_body,
        grid=(_NP // _R,),
        in_specs=[
            pl.BlockSpec((_R, _D), lambda i: (i, 0)),
            pl.BlockSpec((_NC, _R), lambda i: (0, i)),
            pl.BlockSpec((_R, _D), lambda i: (i, 0)),
            pl.BlockSpec((1, _D), lambda i: (0, 0)),
        ],
        out_specs=pl.BlockSpec((_R, _D), lambda i: (i, 0)),
        out_shape=jax.ShapeDtypeStruct((_N, _D), jnp.float32),
    )(accf, dpart, xw, b2)


def kernel(x, edge_index, edge_weight, W, b):
    row = edge_index[0].astype(jnp.int32)
    col = edge_index[1].astype(jnp.int32)
    ew = edge_weight.astype(jnp.float32)
    pad = _EPAD - _E
    row_p = jnp.pad(row, (0, pad))
    col_p = jnp.pad(col, (0, pad))
    ew_p = jnp.pad(ew, (0, pad))

    dpart = _deg_kernel(col_p, ew_p)
    y, xw = _mm_call(x, W, dpart)
    y2 = jnp.stack([y[:, :_DH], y[:, _DH:]],
                   axis=0).reshape(_NC, _NPAIR, _D)
    rp_p = (row_p & 1).astype(jnp.float32)
    cp_p = (col_p & 1).astype(jnp.float32)
    acc2 = _prop_kernel(y2, row_p >> 1, col_p >> 1, ew_p, rp_p, cp_p)
    accf = acc2.reshape(_NC, _NPAIR, 2, _DH).transpose(
        (1, 2, 0, 3)).reshape(_NP, _D)
    return _fin_call(accf, dpart, xw, b.reshape(1, _D))
